# baseline (speedup 1.0000x reference)
import functools

import jax
import jax.numpy as jnp
from jax import lax
from jax.experimental import pallas as pl
from jax.experimental.pallas import tpu as pltpu
from jax.experimental.pallas import tpu_sc as plsc

N = 10000
D = 128
G = 64

NC = 2
NS = 16
NW = NC * NS
L = 16

NSC = 6144
CH = 64
NCH = 3
RPW = CH * NCH
ZR = 8
TB = 2048
NTCB = (N - NSC + TB - 1) // TB
BIR = 80


def _sc_partial_sums(x, bi):
    mesh = plsc.VectorSubcoreMesh(
        core_axis_name="c", subcore_axis_name="s", num_cores=NC, num_subcores=NS
    )

    @functools.partial(
        pl.kernel,
        mesh=mesh,
        out_type=jax.ShapeDtypeStruct((NC, G, D), jnp.float32),
        scratch_types=[
            pltpu.VMEM((NCH, CH, D), jnp.float32),
            pltpu.VMEM((RPW,), jnp.int32),
            pltpu.VMEM((NCH, CH), jnp.int32),
            pltpu.VMEM((ZR, D), jnp.float32),
            pltpu.VMEM_SHARED((G, D), jnp.float32),
            pltpu.SemaphoreType.DMA((NCH,)),
            pltpu.SemaphoreType.DMA,
        ],
    )
    def k(x_hbm, bi_hbm, sums_out, xb, segf, segb, zb, accs, gsem, ssem):
        cid = lax.axis_index("c")
        sid = lax.axis_index("s")
        wid = sid * NC + cid
        base = pl.multiple_of(wid * RPW, CH)

        gathers = []
        for j in range(NCH):
            off = pl.multiple_of(base + j * CH, CH)
            cp = pltpu.async_copy(x_hbm.at[pl.ds(off, CH)], xb.at[j], gsem.at[j])
            gathers.append(cp)
        pltpu.sync_copy(bi_hbm.at[pl.ds(base, RPW)], segf)

        for j in range(NCH):
            for t in range(CH // L):
                segb[j, pl.ds(t * L, L)] = segf[pl.ds(j * CH + t * L, L)]

        zv = jnp.zeros((L,), jnp.float32)
        for i in range(ZR):
            for t in range(D // L):
                zb[i, pl.ds(t * L, L)] = zv

        @pl.when(sid < G // ZR)
        def _():
            off = pl.multiple_of(sid * ZR, ZR)
            pltpu.sync_copy(zb, accs.at[pl.ds(off, ZR)])

        plsc.subcore_barrier()

        scatters = []
        for j in range(NCH):
            gathers[j].wait()
            cp = pltpu.async_copy(xb.at[j], accs.at[segb.at[j]], ssem, add=True)
            scatters.append(cp)
        for cp in scatters:
            cp.wait()

        plsc.subcore_barrier()

        @pl.when(sid == 0)
        def _():
            pltpu.sync_copy(accs, sums_out.at[cid])

    return k(x, bi)


def _tc_partial_sums(x, bi_pad):
    def body(x_ref, bi_ref, o_ref):
        i = pl.program_id(0)

        @pl.when(i == 0)
        def _():
            o_ref[...] = jnp.zeros_like(o_ref)

        ids = lax.broadcasted_iota(jnp.int32, (G, 1), 0)
        oh = jnp.concatenate(
            [
                (bi_ref[k : k + 1] == ids).astype(jnp.float32)
                for k in range(TB // D)
            ],
            axis=1,
        )
        row = NSC + i * TB + lax.broadcasted_iota(jnp.int32, (TB, 1), 0)
        xm = jnp.where(row < N, x_ref[...], 0.0)
        o_ref[...] += jnp.dot(oh, xm, preferred_element_type=jnp.float32)

    return pl.pallas_call(
        body,
        grid=(NTCB,),
        in_specs=[
            pl.BlockSpec((TB, D), lambda i: (NSC // TB + i, 0)),
            pl.BlockSpec((TB // D, D), lambda i: (NSC // TB + i, 0)),
        ],
        out_specs=pl.BlockSpec((G, D), lambda i: (0, 0)),
        out_shape=jax.ShapeDtypeStruct((G, D), jnp.float32),
    )(x, bi_pad)


def _tc_head(psums_sc, psum_tc, bi_pad, W1, b1, W2, b2):
    def body(ps_ref, pt_ref, bi_ref, w1_ref, b1_ref, w2_ref, b2_ref, o_ref):
        sums = ps_ref[0] + ps_ref[1] + pt_ref[...]
        bi = bi_ref[...]
        ids = lax.broadcasted_iota(jnp.int32, (G, 1, 1), 0)
        cnt = jnp.sum((bi[None] == ids).astype(jnp.float32), axis=(1, 2))
        pooled = sums / jnp.maximum(cnt, 1.0)[:, None]
        h = jnp.dot(pooled, w1_ref[...], preferred_element_type=jnp.float32)
        h = jnp.maximum(h + b1_ref[...], 0.0)
        h = jnp.dot(h, w2_ref[...], preferred_element_type=jnp.float32)
        o_ref[...] = jnp.maximum(h + b2_ref[...], 0.0)

    return pl.pallas_call(
        body,
        out_shape=jax.ShapeDtypeStruct((G, D), jnp.float32),
    )(psums_sc, psum_tc, bi_pad, W1, b1.reshape(1, D), W2, b2.reshape(1, D))


def kernel(x, edge_index, edge_attr, batch_index, W1, b1, W2, b2):
    del edge_index, edge_attr
    bi = batch_index.astype(jnp.int32)
    bi_pad = jnp.concatenate(
        [bi, jnp.full((BIR * D - N,), G, jnp.int32)]
    ).reshape(BIR, D)
    psums_sc = _sc_partial_sums(x, bi)
    psum_tc = _tc_partial_sums(x, bi_pad)
    return _tc_head(psums_sc, psum_tc, bi_pad, W1, b1, W2, b2)


# speedup vs baseline: 1.0098x; 1.0098x over previous
import functools

import jax
import jax.numpy as jnp
from jax import lax
from jax.experimental import pallas as pl
from jax.experimental.pallas import tpu as pltpu
from jax.experimental.pallas import tpu_sc as plsc

N = 10000
D = 128
G = 64

NC = 2
NS = 16
NW = NC * NS
L = 16

NSC = 6144
CH = 64
NCH = 3
RPW = CH * NCH
ZR = 8
TB = 1024
NTCB = (N - NSC + TB - 1) // TB
BIR = 80


def _sc_partial_sums(x, bi):
    mesh = plsc.VectorSubcoreMesh(
        core_axis_name="c", subcore_axis_name="s", num_cores=NC, num_subcores=NS
    )

    @functools.partial(
        pl.kernel,
        mesh=mesh,
        out_type=jax.ShapeDtypeStruct((NC, G, D), jnp.float32),
        scratch_types=[
            pltpu.VMEM((NCH, CH, D), jnp.float32),
            pltpu.VMEM((RPW,), jnp.int32),
            pltpu.VMEM((NCH, CH), jnp.int32),
            pltpu.VMEM((ZR, D), jnp.float32),
            pltpu.VMEM_SHARED((G, D), jnp.float32),
            pltpu.SemaphoreType.DMA((NCH,)),
            pltpu.SemaphoreType.DMA,
        ],
    )
    def k(x_hbm, bi_hbm, sums_out, xb, segf, segb, zb, accs, gsem, ssem):
        cid = lax.axis_index("c")
        sid = lax.axis_index("s")
        wid = sid * NC + cid
        base = pl.multiple_of(wid * RPW, CH)

        gathers = []
        for j in range(NCH):
            off = pl.multiple_of(base + j * CH, CH)
            cp = pltpu.async_copy(x_hbm.at[pl.ds(off, CH)], xb.at[j], gsem.at[j])
            gathers.append(cp)
        pltpu.sync_copy(bi_hbm.at[pl.ds(base, RPW)], segf)

        for j in range(NCH):
            for t in range(CH // L):
                segb[j, pl.ds(t * L, L)] = segf[pl.ds(j * CH + t * L, L)]

        zv = jnp.zeros((L,), jnp.float32)
        for i in range(ZR):
            for t in range(D // L):
                zb[i, pl.ds(t * L, L)] = zv

        @pl.when(sid < G // ZR)
        def _():
            off = pl.multiple_of(sid * ZR, ZR)
            pltpu.sync_copy(zb, accs.at[pl.ds(off, ZR)])

        plsc.subcore_barrier()

        scatters = []
        for j in range(NCH):
            gathers[j].wait()
            cp = pltpu.async_copy(xb.at[j], accs.at[segb.at[j]], ssem, add=True)
            scatters.append(cp)
        for cp in scatters:
            cp.wait()

        plsc.subcore_barrier()

        @pl.when(sid == 0)
        def _():
            pltpu.sync_copy(accs, sums_out.at[cid])

    return k(x, bi)


def _tc_partial_sums(x, bi_pad):
    def body(x_ref, bi_ref, o_ref):
        i = pl.program_id(0)

        @pl.when(i == 0)
        def _():
            o_ref[...] = jnp.zeros_like(o_ref)

        ids = lax.broadcasted_iota(jnp.int32, (G, 1), 0)
        oh = jnp.concatenate(
            [
                (bi_ref[k : k + 1] == ids).astype(jnp.float32)
                for k in range(TB // D)
            ],
            axis=1,
        )
        row = NSC + i * TB + lax.broadcasted_iota(jnp.int32, (TB, 1), 0)
        xm = jnp.where(row < N, x_ref[...], 0.0)
        o_ref[...] += jnp.dot(oh, xm, preferred_element_type=jnp.float32)

    return pl.pallas_call(
        body,
        grid=(NTCB,),
        in_specs=[
            pl.BlockSpec((TB, D), lambda i: (NSC // TB + i, 0)),
            pl.BlockSpec((TB // D, D), lambda i: (NSC // TB + i, 0)),
        ],
        out_specs=pl.BlockSpec((G, D), lambda i: (0, 0)),
        out_shape=jax.ShapeDtypeStruct((G, D), jnp.float32),
    )(x, bi_pad)


def _tc_head(psums_sc, psum_tc, bi_pad, W1, b1, W2, b2):
    def body(ps_ref, pt_ref, bi_ref, w1_ref, b1_ref, w2_ref, b2_ref, o_ref):
        sums = ps_ref[0] + ps_ref[1] + pt_ref[...]
        bi = bi_ref[...]
        ids = lax.broadcasted_iota(jnp.int32, (G, 1, 1), 0)
        cnt = jnp.sum((bi[None] == ids).astype(jnp.float32), axis=(1, 2))
        pooled = sums / jnp.maximum(cnt, 1.0)[:, None]
        h = jnp.dot(pooled, w1_ref[...], preferred_element_type=jnp.float32)
        h = jnp.maximum(h + b1_ref[...], 0.0)
        h = jnp.dot(h, w2_ref[...], preferred_element_type=jnp.float32)
        o_ref[...] = jnp.maximum(h + b2_ref[...], 0.0)

    return pl.pallas_call(
        body,
        out_shape=jax.ShapeDtypeStruct((G, D), jnp.float32),
    )(psums_sc, psum_tc, bi_pad, W1, b1.reshape(1, D), W2, b2.reshape(1, D))


def kernel(x, edge_index, edge_attr, batch_index, W1, b1, W2, b2):
    del edge_index, edge_attr
    bi1d = jnp.concatenate(
        [batch_index.astype(jnp.int32), jnp.full((BIR * D - N,), G, jnp.int32)]
    )
    bi_pad = bi1d.reshape(BIR, D)
    psums_sc = _sc_partial_sums(x, bi1d)
    psum_tc = _tc_partial_sums(x, bi_pad)
    return _tc_head(psums_sc, psum_tc, bi_pad, W1, b1, W2, b2)


# speedup vs baseline: 1.0130x; 1.0031x over previous
import functools

import jax
import jax.numpy as jnp
from jax import lax
from jax.experimental import pallas as pl
from jax.experimental.pallas import tpu as pltpu
from jax.experimental.pallas import tpu_sc as plsc

N = 10000
D = 128
G = 64

NC = 2
NS = 16
NW = NC * NS
L = 16

NSC = 6144
CH = 64
NCH = 3
RPW = CH * NCH
ZR = 8
TB = 1024
NTCB = (N - NSC + TB - 1) // TB
BIR = 80


def _sc_partial_sums(x, bi):
    mesh = plsc.VectorSubcoreMesh(
        core_axis_name="c", subcore_axis_name="s", num_cores=NC, num_subcores=NS
    )

    @functools.partial(
        pl.kernel,
        mesh=mesh,
        out_type=jax.ShapeDtypeStruct((NC, G, D), jnp.float32),
        scratch_types=[
            pltpu.VMEM((NCH, CH, D), jnp.float32),
            pltpu.VMEM((RPW,), jnp.int32),
            pltpu.VMEM((NCH, CH), jnp.int32),
            pltpu.VMEM((ZR, D), jnp.float32),
            pltpu.VMEM_SHARED((G, D), jnp.float32),
            pltpu.SemaphoreType.DMA((NCH,)),
            pltpu.SemaphoreType.DMA,
        ],
    )
    def k(x_hbm, bi_hbm, sums_out, xb, segf, segb, zb, accs, gsem, ssem):
        cid = lax.axis_index("c")
        sid = lax.axis_index("s")
        wid = sid * NC + cid
        base = pl.multiple_of(wid * RPW, CH)

        gathers = []
        for j in range(NCH):
            off = pl.multiple_of(base + j * CH, CH)
            cp = pltpu.async_copy(x_hbm.at[pl.ds(off, CH)], xb.at[j], gsem.at[j])
            gathers.append(cp)
        pltpu.sync_copy(bi_hbm.at[pl.ds(base, RPW)], segf)

        for j in range(NCH):
            for t in range(CH // L):
                segb[j, pl.ds(t * L, L)] = segf[pl.ds(j * CH + t * L, L)]

        zv = jnp.zeros((L,), jnp.float32)
        for i in range(ZR):
            for t in range(D // L):
                zb[i, pl.ds(t * L, L)] = zv

        @pl.when(sid < G // ZR)
        def _():
            off = pl.multiple_of(sid * ZR, ZR)
            pltpu.sync_copy(zb, accs.at[pl.ds(off, ZR)])

        plsc.subcore_barrier()

        scatters = []
        for j in range(NCH):
            gathers[j].wait()
            cp = pltpu.async_copy(xb.at[j], accs.at[segb.at[j]], ssem, add=True)
            scatters.append(cp)
        for cp in scatters:
            cp.wait()

        plsc.subcore_barrier()

        @pl.when(sid == 0)
        def _():
            pltpu.sync_copy(accs, sums_out.at[cid])

    return k(x, bi)


def _tc_partial_sums(x, bi_pad):
    def body(x_ref, bi_ref, o_ref):
        i = pl.program_id(0)

        @pl.when(i == 0)
        def _():
            o_ref[...] = jnp.zeros_like(o_ref)

        ids = lax.broadcasted_iota(jnp.int32, (G, 1), 0)
        oh = jnp.concatenate(
            [
                (bi_ref[k : k + 1] == ids).astype(jnp.float32)
                for k in range(TB // D)
            ],
            axis=1,
        )
        row = NSC + i * TB + lax.broadcasted_iota(jnp.int32, (TB, 1), 0)
        xm = jnp.where(row < N, x_ref[...], 0.0)
        o_ref[...] += jnp.dot(oh, xm, preferred_element_type=jnp.float32)

    return pl.pallas_call(
        body,
        grid=(NTCB,),
        in_specs=[
            pl.BlockSpec((TB, D), lambda i: (NSC // TB + i, 0)),
            pl.BlockSpec((TB // D, D), lambda i: (NSC // TB + i, 0)),
        ],
        out_specs=pl.BlockSpec((G, D), lambda i: (0, 0)),
        out_shape=jax.ShapeDtypeStruct((G, D), jnp.float32),
    )(x, bi_pad)


def _tc_head(psums_sc, psum_tc, bi_pad, W1, b1, W2, b2):
    def body(ps_ref, pt_ref, bi_ref, w1_ref, b1_ref, w2_ref, b2_ref, o_ref):
        sums = ps_ref[0] + ps_ref[1] + pt_ref[...]
        bi = bi_ref[...]
        ids = lax.broadcasted_iota(jnp.int32, (G, 1, 1), 0)
        cnt = jnp.sum((bi[None] == ids).astype(jnp.float32), axis=(1, 2))
        pooled = sums / jnp.maximum(cnt, 1.0)[:, None]
        h = jnp.dot(pooled, w1_ref[...], preferred_element_type=jnp.float32)
        h = jnp.maximum(h + b1_ref[...], 0.0)
        h = jnp.dot(h, w2_ref[...], preferred_element_type=jnp.float32)
        o_ref[...] = jnp.maximum(h + b2_ref[...], 0.0)

    return pl.pallas_call(
        body,
        out_shape=jax.ShapeDtypeStruct((G, D), jnp.float32),
    )(psums_sc, psum_tc, bi_pad, W1, b1.reshape(1, D), W2, b2.reshape(1, D))


def kernel(x, edge_index, edge_attr, batch_index, W1, b1, W2, b2):
    del edge_index, edge_attr
    bi = batch_index.astype(jnp.int32)
    bi_pad = jnp.concatenate(
        [bi, jnp.full((BIR * D - N,), G, jnp.int32)]
    ).reshape(BIR, D)
    psums_sc = _sc_partial_sums(x, bi)
    psum_tc = _tc_partial_sums(x, bi_pad)
    return _tc_head(psums_sc, psum_tc, bi_pad, W1, b1, W2, b2)
